# baseline (speedup 1.0000x reference)
import functools

import jax
import jax.numpy as jnp
from jax import lax
from jax.experimental import pallas as pl
from jax.experimental.pallas import tpu as pltpu
from jax.experimental.pallas import tpu_sc as plsc

NUM_CORES = 2
NUM_SUBCORES = 16
NUM_WORKERS = NUM_CORES * NUM_SUBCORES

V, D = 1000, 64
A, W = 4096, 50
A_PER_W = A // NUM_WORKERS
LANES = 16
NCHUNK = A_PER_W // LANES


def _norm_body(counts_ref, prob_ref):
    c = counts_ref[...]
    s = jnp.sum(c, axis=1, keepdims=True)
    prob_ref[...] = c / jnp.maximum(s, 1.0)


_normalize = pl.pallas_call(
    _norm_body,
    out_shape=jax.ShapeDtypeStruct((V, D), jnp.float32),
)


def _gather_body(xt_hbm, tab_hbm, out_hbm, idx_v, tab_v, tbuf_v, *sems):
    wid = lax.axis_index("s") * NUM_CORES + lax.axis_index("c")
    a0 = wid * A_PER_W
    pltpu.sync_copy(xt_hbm.at[:, pl.ds(a0, A_PER_W)], idx_v)
    pltpu.sync_copy(tab_hbm, tab_v)

    def compute(w, s):
        row = idx_v.at[w]
        for k in range(NCHUNK):
            x16 = row[pl.ds(k * LANES, LANES)]
            base = x16 * D
            for d in range(D):
                val = plsc.load_gather(tab_v, [base + d])
                tbuf_v[s, d, pl.ds(k * LANES, LANES)] = val

    def out_slice(w):
        return out_hbm.at[w, :, pl.ds(a0, A_PER_W)]

    def start_write(w, s):
        pltpu.async_copy(tbuf_v.at[s], out_slice(w), sems[s])

    def wait_write(w, s):
        pltpu.make_async_copy(tbuf_v.at[s], out_slice(w), sems[s]).wait()

    compute(0, 0)
    start_write(0, 0)
    compute(1, 1)
    start_write(1, 1)

    def turn(g, carry):
        w0 = 2 * g
        for b in range(2):
            wait_write(w0 - 2 + b, b)
            compute(w0 + b, b)
            start_write(w0 + b, b)
        return carry

    lax.fori_loop(1, W // 2, turn, 0)
    wait_write(W - 2, 0)
    wait_write(W - 1, 1)


_gather = functools.partial(
    pl.kernel,
    mesh=plsc.VectorSubcoreMesh(core_axis_name="c", subcore_axis_name="s"),
    out_type=jax.ShapeDtypeStruct((W, D, A), jnp.float32),
    scratch_types=(
        [
            pltpu.VMEM((W, A_PER_W), jnp.int32),
            pltpu.VMEM((V * D,), jnp.float32),
            pltpu.VMEM((2, D, A_PER_W), jnp.float32),
        ]
        + [pltpu.SemaphoreType.DMA] * 2
    ),
    compiler_params=pltpu.CompilerParams(
        use_tc_tiling_on_sc=True, needs_layout_passes=False
    ),
)(_gather_body)


def kernel(x, pron_counts):
    prob_flat = _normalize(pron_counts).reshape(V * D)
    out_t = _gather(x.T, prob_flat)
    return out_t.transpose(2, 0, 1)


# speedup vs baseline: 2.3961x; 2.3961x over previous
import functools

import jax
import jax.numpy as jnp
from jax import lax
from jax.experimental import pallas as pl
from jax.experimental.pallas import tpu as pltpu
from jax.experimental.pallas import tpu_sc as plsc

NUM_CORES = 2
NUM_SUBCORES = 16
NUM_WORKERS = NUM_CORES * NUM_SUBCORES

V, D = 1000, 64
A, W = 4096, 50
A_PER_W = A // NUM_WORKERS
LANES = 16
NCHUNK = A_PER_W // LANES


PD = 65


def _norm_body(counts_ref, prob_ref):
    c = counts_ref[...]
    s = jnp.sum(c, axis=1, keepdims=True)
    prob_ref[:, :D] = c / jnp.maximum(s, 1.0)


_normalize = pl.pallas_call(
    _norm_body,
    out_shape=jax.ShapeDtypeStruct((V, PD), jnp.float32),
)


def _gather_body(xt_hbm, tab_hbm, out_hbm, idx_v, tab_v, tbuf_v, *sems):
    wid = lax.axis_index("s") * NUM_CORES + lax.axis_index("c")
    a0 = wid * A_PER_W
    pltpu.sync_copy(xt_hbm.at[:, pl.ds(a0, A_PER_W)], idx_v)
    pltpu.sync_copy(tab_hbm, tab_v)

    def compute(w, s):
        def kstep(k, carry):
            x16 = idx_v[w, pl.ds(k * LANES, LANES)]
            base = x16 * PD
            for d in range(D):
                val = plsc.load_gather(tab_v, [base + d])
                tbuf_v[s, d, pl.ds(k * LANES, LANES)] = val
            return carry

        lax.fori_loop(0, NCHUNK, kstep, 0)

    def out_slice(w):
        return out_hbm.at[w, :, pl.ds(a0, A_PER_W)]

    def start_write(w, s):
        pltpu.async_copy(tbuf_v.at[s], out_slice(w), sems[s])

    def wait_write(w, s):
        pltpu.make_async_copy(tbuf_v.at[s], out_slice(w), sems[s]).wait()

    compute(0, 0)
    start_write(0, 0)
    compute(1, 1)
    start_write(1, 1)

    def turn(g, carry):
        w0 = 2 * g
        for b in range(2):
            wait_write(w0 - 2 + b, b)
            compute(w0 + b, b)
            start_write(w0 + b, b)
        return carry

    lax.fori_loop(1, W // 2, turn, 0)
    wait_write(W - 2, 0)
    wait_write(W - 1, 1)


_gather = functools.partial(
    pl.kernel,
    mesh=plsc.VectorSubcoreMesh(core_axis_name="c", subcore_axis_name="s"),
    out_type=jax.ShapeDtypeStruct((W, D, A), jnp.float32),
    scratch_types=(
        [
            pltpu.VMEM((W, A_PER_W), jnp.int32),
            pltpu.VMEM((V * PD,), jnp.float32),
            pltpu.VMEM((2, D, A_PER_W), jnp.float32),
        ]
        + [pltpu.SemaphoreType.DMA] * 2
    ),
    compiler_params=pltpu.CompilerParams(
        use_tc_tiling_on_sc=True, needs_layout_passes=False
    ),
)(_gather_body)


def kernel(x, pron_counts):
    prob_flat = _normalize(pron_counts).reshape(V * PD)
    out_t = _gather(x.T, prob_flat)
    return out_t.transpose(2, 0, 1)


# speedup vs baseline: 5.4583x; 2.2780x over previous
import functools

import jax
import jax.numpy as jnp
from jax import lax
from jax.experimental import pallas as pl
from jax.experimental.pallas import tpu as pltpu
from jax.experimental.pallas import tpu_sc as plsc

NUM_CORES = 2
NUM_SUBCORES = 16
NUM_WORKERS = NUM_CORES * NUM_SUBCORES

V, D = 1000, 64
A, W = 4096, 50
A_PER_W = A // NUM_WORKERS
LANES = 16
NCHUNK = A_PER_W // LANES


PD = 65


def _norm_body(counts_ref, prob_ref):
    c = counts_ref[...]
    s = jnp.sum(c, axis=1, keepdims=True)
    prob_ref[:, :D] = c / jnp.maximum(s, 1.0)


_normalize = pl.pallas_call(
    _norm_body,
    out_shape=jax.ShapeDtypeStruct((V, PD), jnp.float32),
)


def _gather_body(xt_hbm, tab_hbm, out_hbm, idx_v, tab_v, tbuf_v, *sems):
    wid = lax.axis_index("s") * NUM_CORES + lax.axis_index("c")
    a0 = wid * A_PER_W
    pltpu.sync_copy(xt_hbm.at[:, pl.ds(a0, A_PER_W)], idx_v)
    pltpu.sync_copy(tab_hbm, tab_v)

    def compute(w, s):
        def kstep(k, carry):
            x16 = idx_v[w, pl.ds(k * LANES, LANES)]
            base = x16 * PD
            G = 8
            for d0 in range(0, D, G):
                vals = [
                    plsc.load_gather(tab_v, [base + (d0 + j)]) for j in range(G)
                ]
                for j in range(G):
                    tbuf_v[s, d0 + j, pl.ds(k * LANES, LANES)] = vals[j]
            return carry

        lax.fori_loop(0, NCHUNK, kstep, 0)

    def out_slice(w):
        return out_hbm.at[w, :, pl.ds(a0, A_PER_W)]

    def start_write(w, s):
        pltpu.async_copy(tbuf_v.at[s], out_slice(w), sems[s])

    def wait_write(w, s):
        pltpu.make_async_copy(tbuf_v.at[s], out_slice(w), sems[s]).wait()

    compute(0, 0)
    start_write(0, 0)
    compute(1, 1)
    start_write(1, 1)

    def turn(g, carry):
        w0 = 2 * g
        for b in range(2):
            wait_write(w0 - 2 + b, b)
            compute(w0 + b, b)
            start_write(w0 + b, b)
        return carry

    lax.fori_loop(1, W // 2, turn, 0)
    wait_write(W - 2, 0)
    wait_write(W - 1, 1)


_gather = functools.partial(
    pl.kernel,
    mesh=plsc.VectorSubcoreMesh(core_axis_name="c", subcore_axis_name="s"),
    out_type=jax.ShapeDtypeStruct((W, D, A), jnp.float32),
    scratch_types=(
        [
            pltpu.VMEM((W, A_PER_W), jnp.int32),
            pltpu.VMEM((V * PD,), jnp.float32),
            pltpu.VMEM((2, D, A_PER_W), jnp.float32),
        ]
        + [pltpu.SemaphoreType.DMA] * 2
    ),
    compiler_params=pltpu.CompilerParams(
        use_tc_tiling_on_sc=True, needs_layout_passes=False
    ),
)(_gather_body)


def kernel(x, pron_counts):
    prob_flat = _normalize(pron_counts).reshape(V * PD)
    out_t = _gather(x.T, prob_flat)
    return out_t.transpose(2, 0, 1)


# speedup vs baseline: 5.4827x; 1.0045x over previous
import functools

import jax
import jax.numpy as jnp
from jax import lax
from jax.experimental import pallas as pl
from jax.experimental.pallas import tpu as pltpu
from jax.experimental.pallas import tpu_sc as plsc

NUM_CORES = 2
NUM_SUBCORES = 16
NUM_WORKERS = NUM_CORES * NUM_SUBCORES

V, D = 1000, 64
A, W = 4096, 50
A_PER_W = A // NUM_WORKERS
LANES = 16
NCHUNK = A_PER_W // LANES


PD = 65


def _norm_body(counts_ref, prob_ref):
    c = counts_ref[...]
    s = jnp.sum(c, axis=1, keepdims=True)
    prob_ref[:, :D] = c / jnp.maximum(s, 1.0)


_normalize = pl.pallas_call(
    _norm_body,
    out_shape=jax.ShapeDtypeStruct((V, PD), jnp.float32),
)


def _gather_body(xt_hbm, tab_hbm, out_hbm, idx_v, tab_v, tbuf_v, *sems):
    wid = lax.axis_index("s") * NUM_CORES + lax.axis_index("c")
    a0 = wid * A_PER_W
    cp_idx = pltpu.async_copy(xt_hbm.at[:, pl.ds(a0, A_PER_W)], idx_v, sems[0])
    cp_tab = pltpu.async_copy(tab_hbm, tab_v, sems[1])
    cp_idx.wait()
    cp_tab.wait()

    def compute(w, s):
        def kstep(k, carry):
            x16 = idx_v[w, pl.ds(k * LANES, LANES)]
            base = x16 * PD
            G = 16
            for d0 in range(0, D, G):
                vals = [
                    plsc.load_gather(tab_v, [base + (d0 + j)]) for j in range(G)
                ]
                for j in range(G):
                    tbuf_v[s, d0 + j, pl.ds(k * LANES, LANES)] = vals[j]
            return carry

        lax.fori_loop(0, NCHUNK, kstep, 0)

    def out_slice(w):
        return out_hbm.at[w, :, pl.ds(a0, A_PER_W)]

    def start_write(w, s):
        pltpu.async_copy(tbuf_v.at[s], out_slice(w), sems[s])

    def wait_write(w, s):
        pltpu.make_async_copy(tbuf_v.at[s], out_slice(w), sems[s]).wait()

    compute(0, 0)
    start_write(0, 0)
    compute(1, 1)
    start_write(1, 1)

    def turn(g, carry):
        w0 = 2 * g
        for b in range(2):
            wait_write(w0 - 2 + b, b)
            compute(w0 + b, b)
            start_write(w0 + b, b)
        return carry

    lax.fori_loop(1, W // 2, turn, 0)
    wait_write(W - 2, 0)
    wait_write(W - 1, 1)


_gather = functools.partial(
    pl.kernel,
    mesh=plsc.VectorSubcoreMesh(core_axis_name="c", subcore_axis_name="s"),
    out_type=jax.ShapeDtypeStruct((W, D, A), jnp.float32),
    scratch_types=(
        [
            pltpu.VMEM((W, A_PER_W), jnp.int32),
            pltpu.VMEM((V * PD,), jnp.float32),
            pltpu.VMEM((2, D, A_PER_W), jnp.float32),
        ]
        + [pltpu.SemaphoreType.DMA] * 2
    ),
    compiler_params=pltpu.CompilerParams(
        use_tc_tiling_on_sc=True, needs_layout_passes=False
    ),
)(_gather_body)


def kernel(x, pron_counts):
    prob_flat = _normalize(pron_counts).reshape(V * PD)
    out_t = _gather(x.T, prob_flat)
    return out_t.transpose(2, 0, 1)


# speedup vs baseline: 7.3617x; 1.3427x over previous
import functools

import jax
import jax.numpy as jnp
from jax import lax
from jax.experimental import pallas as pl
from jax.experimental.pallas import tpu as pltpu
from jax.experimental.pallas import tpu_sc as plsc

NUM_CORES = 2
NUM_SUBCORES = 16
NUM_WORKERS = NUM_CORES * NUM_SUBCORES

V, D = 1000, 64
A, W = 4096, 50
A_PER_W = A // NUM_WORKERS
LANES = 16
NCHUNK = A_PER_W // LANES


PD = 65


def _norm_body(counts_ref, prob_ref):
    c = counts_ref[...]
    s = jnp.sum(c, axis=1, keepdims=True)
    prob_ref[:, :D] = c / jnp.maximum(s, 1.0)


_normalize = pl.pallas_call(
    _norm_body,
    out_shape=jax.ShapeDtypeStruct((V, PD), jnp.float32),
)


def _gather_body(xt_hbm, tab_hbm, out_hbm, idx_v, tab_v, tbuf_v, *sems):
    wid = lax.axis_index("s") * NUM_CORES + lax.axis_index("c")
    a0 = wid * A_PER_W
    cp_idx = pltpu.async_copy(xt_hbm.at[:, pl.ds(a0, A_PER_W)], idx_v, sems[0])
    cp_tab = pltpu.async_copy(tab_hbm, tab_v, sems[1])
    cp_idx.wait()
    cp_tab.wait()

    def compute(w, s):
        @functools.partial(plsc.parallel_loop, 0, NCHUNK)
        def kstep(k):
            x16 = idx_v[w, pl.ds(k * LANES, LANES)]
            base = x16 * PD
            G = 16
            for d0 in range(0, D, G):
                vals = [
                    plsc.load_gather(tab_v, [base + (d0 + j)]) for j in range(G)
                ]
                for j in range(G):
                    tbuf_v[s, d0 + j, pl.ds(k * LANES, LANES)] = vals[j]

    def out_slice(w):
        return out_hbm.at[w, :, pl.ds(a0, A_PER_W)]

    def start_write(w, s):
        pltpu.async_copy(tbuf_v.at[s], out_slice(w), sems[s])

    def wait_write(w, s):
        pltpu.make_async_copy(tbuf_v.at[s], out_slice(w), sems[s]).wait()

    compute(0, 0)
    start_write(0, 0)
    compute(1, 1)
    start_write(1, 1)

    def turn(g, carry):
        w0 = 2 * g
        for b in range(2):
            wait_write(w0 - 2 + b, b)
            compute(w0 + b, b)
            start_write(w0 + b, b)
        return carry

    lax.fori_loop(1, W // 2, turn, 0)
    wait_write(W - 2, 0)
    wait_write(W - 1, 1)


_gather = functools.partial(
    pl.kernel,
    mesh=plsc.VectorSubcoreMesh(core_axis_name="c", subcore_axis_name="s"),
    out_type=jax.ShapeDtypeStruct((W, D, A), jnp.float32),
    scratch_types=(
        [
            pltpu.VMEM((W, A_PER_W), jnp.int32),
            pltpu.VMEM((V * PD,), jnp.float32),
            pltpu.VMEM((2, D, A_PER_W), jnp.float32),
        ]
        + [pltpu.SemaphoreType.DMA] * 2
    ),
    compiler_params=pltpu.CompilerParams(
        use_tc_tiling_on_sc=True, needs_layout_passes=False
    ),
)(_gather_body)


def kernel(x, pron_counts):
    prob_flat = _normalize(pron_counts).reshape(V * PD)
    out_t = _gather(x.T, prob_flat)
    return out_t.transpose(2, 0, 1)
